# baseline (speedup 1.0000x reference)
import jax
import jax.numpy as jnp
from jax import lax
from jax.experimental import pallas as pl
from jax.experimental.pallas import tpu as pltpu
from jax.experimental.pallas import tpu_sc as plsc

_N = 16384
_L = 20
_D = 32
_HALF = 16

_NC = 2
_NS = 16
_NW = _NC * _NS
_NODES_PER_W = _N // _NW
_C = 64
_CL = _C * _L
_CHUNKS = _NODES_PER_W // _C


def _sc_body(type_idx_hbm, sub_ids_hbm, type_table_hbm, token_table_hbm,
             out_hbm, idx_a, idx_b, tidx_a, tidx_b,
             tok_a, tok_b, typ_a, typ_b, out_a, out_b,
             sem_a, sem_b, semt_a, semt_b):
    wid = lax.axis_index("s") * _NC + lax.axis_index("c")

    idx_bufs = (idx_a, idx_b)
    tidx_bufs = (tidx_a, tidx_b)
    tok_bufs = (tok_a, tok_b)
    typ_bufs = (typ_a, typ_b)
    out_bufs = (out_a, out_b)
    sems = (sem_a, sem_b)
    semts = (semt_a, semt_b)

    tok_base = wid * _NODES_PER_W * _L
    typ_base = wid * _NODES_PER_W

    def fire(c):
        b = c % 2
        pltpu.sync_copy(sub_ids_hbm.at[pl.ds(tok_base + c * _CL, _CL)],
                        idx_bufs[b])
        pltpu.sync_copy(type_idx_hbm.at[pl.ds(typ_base + c * _C, _C)],
                        tidx_bufs[b])
        d_tok = pltpu.async_copy(token_table_hbm.at[idx_bufs[b]],
                                 tok_bufs[b], sems[b])
        d_typ = pltpu.async_copy(type_table_hbm.at[tidx_bufs[b]],
                                 typ_bufs[b], semts[b])
        return (d_tok, d_typ)

    descs = fire(0)
    for c in range(_CHUNKS):
        b = c % 2
        next_descs = fire(c + 1) if c + 1 < _CHUNKS else None
        for dsc in descs:
            dsc.wait()
        descs = next_descs

        tok_v = tok_bufs[b]
        typ_v = typ_bufs[b]
        out_v = out_bufs[b]

        @plsc.parallel_loop(0, _C, unroll=2)
        def node_body(n):
            base = n * _L
            acc0 = tok_v[base, pl.ds(0, _HALF)]
            acc1 = tok_v[base, pl.ds(_HALF, _HALF)]
            for l in range(1, _L):
                acc0 = acc0 + tok_v[base + l, pl.ds(0, _HALF)]
                acc1 = acc1 + tok_v[base + l, pl.ds(_HALF, _HALF)]
            o = n * (2 * _D)
            out_v[pl.ds(o, _HALF)] = typ_v[n, pl.ds(0, _HALF)]
            out_v[pl.ds(o + _HALF, _HALF)] = typ_v[n, pl.ds(_HALF, _HALF)]
            out_v[pl.ds(o + 2 * _HALF, _HALF)] = acc0 * (1.0 / _L)
            out_v[pl.ds(o + 3 * _HALF, _HALF)] = acc1 * (1.0 / _L)

        nbase = wid * _NODES_PER_W + c * _C
        pltpu.sync_copy(out_v, out_hbm.at[pl.ds(nbase * 2 * _D, _C * 2 * _D)])


def kernel(node_type_index, node_sub_token_ids, type_table, token_table):
    sub_ids_flat = node_sub_token_ids.reshape(_N * _L)

    mesh = plsc.VectorSubcoreMesh(core_axis_name="c", subcore_axis_name="s")
    run = pl.kernel(
        _sc_body,
        mesh=mesh,
        compiler_params=pltpu.CompilerParams(use_tc_tiling_on_sc=False),
        out_type=jax.ShapeDtypeStruct((_N * 2 * _D,), jnp.float32),
        scratch_types=[
            pltpu.VMEM((_CL,), jnp.int32),
            pltpu.VMEM((_CL,), jnp.int32),
            pltpu.VMEM((_C,), jnp.int32),
            pltpu.VMEM((_C,), jnp.int32),
            pltpu.VMEM((_CL, _D), jnp.float32),
            pltpu.VMEM((_CL, _D), jnp.float32),
            pltpu.VMEM((_C, _D), jnp.float32),
            pltpu.VMEM((_C, _D), jnp.float32),
            pltpu.VMEM((_C * 2 * _D,), jnp.float32),
            pltpu.VMEM((_C * 2 * _D,), jnp.float32),
            pltpu.SemaphoreType.DMA,
            pltpu.SemaphoreType.DMA,
            pltpu.SemaphoreType.DMA,
            pltpu.SemaphoreType.DMA,
        ],
    )
    flat = run(node_type_index, sub_ids_flat, type_table, token_table)
    return flat.reshape(_N, 2 * _D)
